# baseline (speedup 1.0000x reference)
import functools

import jax
import jax.numpy as jnp
from jax import lax
from jax.experimental import pallas as pl
from jax.experimental.pallas import tpu as pltpu
from jax.experimental.pallas import tpu_sc as plsc

_N_ITEMS = 1000000
_N_OUT = 1000
_D = 64
_B = 4096
_NCH = 2
_CB = _B // _NCH
_CK = 8
_CW = 128
_BM = 512


def _make_sc_gather(B, D):
    info = plsc.get_sparse_core_info()
    NC, NS, L = info.num_cores, info.num_subcores, info.num_lanes
    b_per_w = 128
    NW = B // b_per_w
    n_chunks = b_per_w // _CK
    mesh = plsc.VectorSubcoreMesh(core_axis_name="c", subcore_axis_name="s")

    @functools.partial(
        pl.kernel,
        mesh=mesh,
        out_type=jax.ShapeDtypeStruct((D, B), jnp.float32),
        compiler_params=pltpu.CompilerParams(needs_layout_passes=False),
        scratch_types=[
            pltpu.VMEM((b_per_w + L, ), jnp.int32),
            pltpu.VMEM((_CK, D, _CW), jnp.float32),
            pltpu.VMEM((D, b_per_w), jnp.float32),
            pltpu.SemaphoreType.DMA,
        ],
    )
    def gather_k(table_hbm, idx_hbm, out_hbm, idx_v, ring_v, zt_v, sem):
        lane = lax.iota(jnp.int32, L)
        wid = lax.axis_index("s") * NC + lax.axis_index("c")

        @pl.when(wid < NW)
        def _():
            base = wid * b_per_w
            pltpu.sync_copy(
                idx_hbm.at[pl.ds(base, b_per_w)], idx_v.at[pl.ds(0, b_per_w)]
            )

            @pl.loop(0, n_chunks)
            def _(ch):
                k0 = ch * _CK
                ivec = idx_v[pl.ds(k0, L)]
                cvec = (ivec >> 7) << 7
                lvec = ivec & (_CW - 1)

                copies = []
                lscal = []
                for j in range(_CK):
                    c = jnp.sum(jnp.where(lane == j, cvec, 0))
                    lscal.append(jnp.sum(jnp.where(lane == j, lvec, 0)))
                    col0 = pl.multiple_of(c, _CW)
                    copies.append(
                        pltpu.async_copy(
                            table_hbm.at[:, pl.ds(col0, _CW)], ring_v.at[j], sem
                        )
                    )
                for cp in copies:
                    cp.wait()

                for j in range(_CK):
                    lj = lscal[j]
                    for c4 in range(D // L):
                        rows = lane + c4 * L
                        vals = plsc.load_gather(
                            ring_v.at[j], [rows, jnp.full((L,), lj, jnp.int32)]
                        )
                        plsc.store_scatter(
                            zt_v, [rows, jnp.full((L,), k0 + j, jnp.int32)], vals
                        )

            pltpu.sync_copy(zt_v, out_hbm.at[:, pl.ds(base, b_per_w)])

    return gather_k


def _dense_body(w_ref, zt_ref, b_ref, o_ref):
    logits = (
        lax.dot_general(
            w_ref[...], zt_ref[...],
            (((1,), (0,)), ((), ())),
            preferred_element_type=jnp.float32,
        )
        + b_ref[...]
    )
    m = jnp.max(logits, axis=0, keepdims=True)
    e = jnp.exp(logits - m)
    o_ref[...] = e / jnp.sum(e, axis=0, keepdims=True)


def _dense_body_acc(prev_ref, w_ref, zt_ref, b_ref, o_ref):
    del prev_ref
    _dense_body(w_ref, zt_ref, b_ref, o_ref)


def _dense_stripe(w, zt, bcol, chunk, prev=None):
    n_out = w.shape[0]
    steps = _CB // _BM
    out_shape = jax.ShapeDtypeStruct((n_out, _B), jnp.float32)
    common = dict(
        grid=(steps,),
        out_specs=pl.BlockSpec(
            (n_out, _BM), lambda i, c=chunk: (0, c * steps + i)
        ),
        out_shape=out_shape,
    )
    w_spec = pl.BlockSpec((n_out, _D), lambda i: (0, 0))
    zt_spec = pl.BlockSpec((_D, _BM), lambda i: (0, i))
    b_spec = pl.BlockSpec((n_out, 1), lambda i: (0, 0))
    if prev is None:
        return pl.pallas_call(
            _dense_body, in_specs=[w_spec, zt_spec, b_spec], **common
        )(w, zt, bcol)
    return pl.pallas_call(
        _dense_body_acc,
        in_specs=[pl.BlockSpec(memory_space=pl.ANY), w_spec, zt_spec, b_spec],
        input_output_aliases={0: 0},
        **common,
    )(prev, w, zt, bcol)


def kernel(item_ids, emb_table, fc_w, fc_b):
    idx = item_ids.astype(jnp.int32)
    table_t = emb_table.T
    bcol = fc_b.reshape(_N_OUT, 1)
    gather = _make_sc_gather(_CB, _D)
    out_t = None
    for c in range(_NCH):
        zt = gather(table_t, lax.dynamic_slice_in_dim(idx, c * _CB, _CB))
        out_t = _dense_stripe(fc_w, zt, bcol, c, prev=out_t)
    return out_t.T


# speedup vs baseline: 1.4007x; 1.4007x over previous
import functools

import jax
import jax.numpy as jnp
from jax import lax
from jax.experimental import pallas as pl
from jax.experimental.pallas import tpu as pltpu
from jax.experimental.pallas import tpu_sc as plsc

_N_ITEMS = 1000000
_N_OUT = 1000
_D = 64
_B = 4096
_NCH = 2
_CB = _B // _NCH
_CK = 8
_CW = 128
_BM = 512


def _make_sc_gather(B, D):
    info = plsc.get_sparse_core_info()
    NC, NS, L = info.num_cores, info.num_subcores, info.num_lanes
    NW = NC * NS
    b_per_w = B // NW
    n_chunks = b_per_w // _CK
    mesh = plsc.VectorSubcoreMesh(core_axis_name="c", subcore_axis_name="s")

    @functools.partial(
        pl.kernel,
        mesh=mesh,
        out_type=jax.ShapeDtypeStruct((NW, D, b_per_w), jnp.float32),
        compiler_params=pltpu.CompilerParams(needs_layout_passes=False),
        scratch_types=[
            pltpu.VMEM((b_per_w + L, ), jnp.int32),
            pltpu.VMEM((_CK, D, _CW), jnp.float32),
            pltpu.VMEM((D, b_per_w), jnp.float32),
            pltpu.SemaphoreType.DMA,
        ],
    )
    def gather_k(table_hbm, idx_hbm, out_hbm, idx_v, ring_v, zt_v, sem):
        lane = lax.iota(jnp.int32, L)
        wid = lax.axis_index("s") * NC + lax.axis_index("c")

        if True:
            base = wid * b_per_w
            pltpu.sync_copy(
                idx_hbm.at[pl.ds(base, b_per_w)], idx_v.at[pl.ds(0, b_per_w)]
            )

            @pl.loop(0, n_chunks)
            def _(ch):
                k0 = ch * _CK
                ivec = idx_v[pl.ds(k0, L)]
                cvec = (ivec >> 7) << 7
                lvec = ivec & (_CW - 1)

                copies = []
                lscal = []
                for j in range(_CK):
                    c = jnp.sum(jnp.where(lane == j, cvec, 0))
                    lscal.append(jnp.sum(jnp.where(lane == j, lvec, 0)))
                    col0 = pl.multiple_of(c, _CW)
                    copies.append(
                        pltpu.async_copy(
                            table_hbm.at[:, pl.ds(col0, _CW)], ring_v.at[j], sem
                        )
                    )
                for cp in copies:
                    cp.wait()

                for j in range(_CK):
                    lj = lscal[j]
                    for c4 in range(D // L):
                        rows = lane + c4 * L
                        vals = plsc.load_gather(
                            ring_v.at[j], [rows, jnp.full((L,), lj, jnp.int32)]
                        )
                        plsc.store_scatter(
                            zt_v, [rows, jnp.full((L,), k0 + j, jnp.int32)], vals
                        )

            pltpu.sync_copy(zt_v, out_hbm.at[wid])

    return gather_k


def _dense_body(w_ref, zt_ref, b_ref, o_ref):
    logits = (
        lax.dot_general(
            w_ref[...], zt_ref[...],
            (((1,), (0,)), ((), ())),
            preferred_element_type=jnp.float32,
        )
        + b_ref[...]
    )
    m = jnp.max(logits, axis=0, keepdims=True)
    e = jnp.exp(logits - m)
    o_ref[...] = e / jnp.sum(e, axis=0, keepdims=True)


def _dense_body_acc(prev_ref, w_ref, zt_ref, b_ref, o_ref):
    del prev_ref
    _dense_body(w_ref, zt_ref, b_ref, o_ref)


def _dense_stripe(w, zt, bcol, chunk, prev=None):
    n_out = w.shape[0]
    steps = _CB // _BM
    out_shape = jax.ShapeDtypeStruct((n_out, _B), jnp.float32)
    common = dict(
        grid=(steps,),
        out_specs=pl.BlockSpec(
            (n_out, _BM), lambda i, c=chunk: (0, c * steps + i)
        ),
        out_shape=out_shape,
    )
    w_spec = pl.BlockSpec((n_out, _D), lambda i: (0, 0))
    zt_spec = pl.BlockSpec((_D, _BM), lambda i: (0, i))
    b_spec = pl.BlockSpec((n_out, 1), lambda i: (0, 0))
    if prev is None:
        return pl.pallas_call(
            _dense_body, in_specs=[w_spec, zt_spec, b_spec], **common
        )(w, zt, bcol)
    return pl.pallas_call(
        _dense_body_acc,
        in_specs=[pl.BlockSpec(memory_space=pl.ANY), w_spec, zt_spec, b_spec],
        input_output_aliases={0: 0},
        **common,
    )(prev, w, zt, bcol)


def kernel(item_ids, emb_table, fc_w, fc_b):
    idx = item_ids.astype(jnp.int32)
    table_t = emb_table.T
    bcol = fc_b.reshape(_N_OUT, 1)
    gather = _make_sc_gather(_CB, _D)
    out_t = None
    for c in range(_NCH):
        zt3 = gather(table_t, lax.dynamic_slice_in_dim(idx, c * _CB, _CB))
        zt = jnp.transpose(zt3, (1, 0, 2)).reshape(_D, _CB)
        out_t = _dense_stripe(fc_w, zt, bcol, c, prev=out_t)
    return out_t.T


# speedup vs baseline: 1.5964x; 1.1397x over previous
import functools

import jax
import jax.numpy as jnp
from jax import lax
from jax.experimental import pallas as pl
from jax.experimental.pallas import tpu as pltpu
from jax.experimental.pallas import tpu_sc as plsc

_N_ITEMS = 1000000
_N_OUT = 1000
_D = 64
_B = 4096
_CK = 8
_CW = 128


def _make_sc_gather(B, D):
    info = plsc.get_sparse_core_info()
    NC, NS, L = info.num_cores, info.num_subcores, info.num_lanes
    NW = NC * NS
    b_per_w = B // NW
    n_chunks = b_per_w // _CK
    mesh = plsc.VectorSubcoreMesh(core_axis_name="c", subcore_axis_name="s")

    @functools.partial(
        pl.kernel,
        mesh=mesh,
        out_type=jax.ShapeDtypeStruct((D, B), jnp.float32),
        compiler_params=pltpu.CompilerParams(needs_layout_passes=False),
        scratch_types=[
            pltpu.VMEM((b_per_w + L, ), jnp.int32),
            pltpu.VMEM((_CK, D, _CW), jnp.float32),
            pltpu.VMEM((D, b_per_w), jnp.float32),
            pltpu.SemaphoreType.DMA,
        ],
    )
    def gather_k(table_hbm, idx_hbm, out_hbm, idx_v, ring_v, zt_v, sem):
        lane = lax.iota(jnp.int32, L)
        wid = lax.axis_index("s") * NC + lax.axis_index("c")
        base = wid * b_per_w
        pltpu.sync_copy(
            idx_hbm.at[pl.ds(base, b_per_w)], idx_v.at[pl.ds(0, b_per_w)]
        )

        @pl.loop(0, n_chunks)
        def _(ch):
            k0 = ch * _CK
            ivec = idx_v[pl.ds(k0, L)]
            cvec = (ivec >> 7) << 7
            lvec = ivec & (_CW - 1)

            copies = []
            lscal = []
            for j in range(_CK):
                c = jnp.sum(jnp.where(lane == j, cvec, 0))
                lscal.append(jnp.sum(jnp.where(lane == j, lvec, 0)))
                col0 = pl.multiple_of(c, _CW)
                copies.append(
                    pltpu.async_copy(
                        table_hbm.at[:, pl.ds(col0, _CW)], ring_v.at[j], sem
                    )
                )
            for cp in copies:
                cp.wait()

            for j in range(_CK):
                lj = lscal[j]
                for c4 in range(D // L):
                    rows = lane + c4 * L
                    vals = plsc.load_gather(
                        ring_v.at[j], [rows, jnp.full((L,), lj, jnp.int32)]
                    )
                    plsc.store_scatter(
                        zt_v, [rows, jnp.full((L,), k0 + j, jnp.int32)], vals
                    )

        pltpu.sync_copy(zt_v, out_hbm.at[:, pl.ds(base, b_per_w)])

    return gather_k


def _dense_body(w_ref, zt_ref, b_ref, o_ref):
    logits = (
        lax.dot_general(
            w_ref[...], zt_ref[...],
            (((0,), (0,)), ((), ())),
            preferred_element_type=jnp.float32,
        )
        + b_ref[...]
    )
    m = jnp.max(logits, axis=0, keepdims=True)
    e = jnp.exp(logits - m)
    o_ref[...] = e / jnp.sum(e, axis=0, keepdims=True)


def _dense_t(wt, zt, bcol, bm):
    B = zt.shape[1]
    n_out = wt.shape[1]
    return pl.pallas_call(
        _dense_body,
        grid=(B // bm,),
        in_specs=[
            pl.BlockSpec((_D, n_out), lambda i: (0, 0)),
            pl.BlockSpec((_D, bm), lambda i: (0, i)),
            pl.BlockSpec((n_out, 1), lambda i: (0, 0)),
        ],
        out_specs=pl.BlockSpec((n_out, bm), lambda i: (0, i)),
        out_shape=jax.ShapeDtypeStruct((n_out, B), jnp.float32),
    )(wt, zt, bcol)


def kernel(item_ids, emb_table, fc_w, fc_b):
    idx = item_ids.astype(jnp.int32)
    table_t = emb_table.T
    zt = _make_sc_gather(_B, _D)(table_t, idx)
    bcol = fc_b.reshape(_N_OUT, 1)
    out_t = _dense_t(fc_w.T, zt, bcol, bm=512)
    return out_t.T


# speedup vs baseline: 1.6175x; 1.0132x over previous
import functools

import jax
import jax.numpy as jnp
from jax import lax
from jax.experimental import pallas as pl
from jax.experimental.pallas import tpu as pltpu
from jax.experimental.pallas import tpu_sc as plsc

_N_ITEMS = 1000000
_N_OUT = 1000
_D = 64
_B = 4096
_CK = 8
_CW = 128


def _make_sc_gather(B, D):
    info = plsc.get_sparse_core_info()
    NC, NS, L = info.num_cores, info.num_subcores, info.num_lanes
    NW = NC * NS
    b_per_w = B // NW
    n_chunks = b_per_w // _CK
    mesh = plsc.VectorSubcoreMesh(core_axis_name="c", subcore_axis_name="s")

    @functools.partial(
        pl.kernel,
        mesh=mesh,
        out_type=jax.ShapeDtypeStruct((D, B), jnp.float32),
        compiler_params=pltpu.CompilerParams(needs_layout_passes=False),
        scratch_types=[
            pltpu.VMEM((b_per_w + L, ), jnp.int32),
            pltpu.VMEM((_CK, D, _CW), jnp.float32),
            pltpu.VMEM((D, b_per_w), jnp.float32),
            pltpu.SemaphoreType.DMA,
        ],
    )
    def gather_k(table_hbm, idx_hbm, out_hbm, idx_v, ring_v, zt_v, sem):
        lane = lax.iota(jnp.int32, L)
        wid = lax.axis_index("s") * NC + lax.axis_index("c")
        base = wid * b_per_w
        pltpu.sync_copy(
            idx_hbm.at[pl.ds(base, b_per_w)], idx_v.at[pl.ds(0, b_per_w)]
        )

        @pl.loop(0, n_chunks)
        def _(ch):
            k0 = ch * _CK
            ivec = idx_v[pl.ds(k0, L)]
            cvec = (ivec >> 7) << 7
            lvec = ivec & (_CW - 1)

            copies = []
            lscal = []
            for j in range(_CK):
                c = jnp.sum(jnp.where(lane == j, cvec, 0))
                lscal.append(jnp.sum(jnp.where(lane == j, lvec, 0)))
                col0 = pl.multiple_of(c, _CW)
                copies.append(
                    pltpu.async_copy(
                        table_hbm.at[:, pl.ds(col0, _CW)], ring_v.at[j], sem
                    )
                )
            for cp in copies:
                cp.wait()

            for j in range(_CK):
                lj = lscal[j]
                for c4 in range(D // L):
                    rows = lane + c4 * L
                    vals = plsc.load_gather(
                        ring_v.at[j], [rows, jnp.full((L,), lj, jnp.int32)]
                    )
                    plsc.store_scatter(
                        zt_v, [rows, jnp.full((L,), k0 + j, jnp.int32)], vals
                    )

        pltpu.sync_copy(zt_v, out_hbm.at[:, pl.ds(base, b_per_w)])

    return gather_k


def _dense_body(w_ref, zt_ref, b_ref, o_ref):
    logits = (
        lax.dot_general(
            w_ref[...], zt_ref[...],
            (((0,), (0,)), ((), ())),
            preferred_element_type=jnp.float32,
        )
        + b_ref[...]
    )
    m = jnp.max(logits, axis=0, keepdims=True)
    e = jnp.exp(logits - m)
    o_ref[...] = e / jnp.sum(e, axis=0, keepdims=True)


def _dense_t(wt, zt, bcol, bm):
    B = zt.shape[1]
    n_out = wt.shape[1]
    return pl.pallas_call(
        _dense_body,
        grid=(B // bm,),
        in_specs=[
            pl.BlockSpec((_D, n_out), lambda i: (0, 0)),
            pl.BlockSpec((_D, bm), lambda i: (0, i)),
            pl.BlockSpec((n_out, 1), lambda i: (0, 0)),
        ],
        out_specs=pl.BlockSpec((n_out, bm), lambda i: (0, i)),
        out_shape=jax.ShapeDtypeStruct((n_out, B), jnp.float32),
    )(wt, zt, bcol)


def kernel(item_ids, emb_table, fc_w, fc_b):
    idx = item_ids.astype(jnp.int32)
    table_t = emb_table.T
    zt = _make_sc_gather(_B, _D)(table_t, idx)
    bcol = fc_b.reshape(_N_OUT, 1)
    out_t = _dense_t(fc_w.T, zt, bcol, bm=1024)
    return out_t.T
